# baseline (speedup 1.0000x reference)
import functools

import jax
import jax.numpy as jnp
from jax import lax
from jax.experimental import pallas as pl
from jax.experimental.pallas import tpu as pltpu
from jax.experimental.pallas import tpu_sc as plsc

NC = 2
NS = 16
L = 16
NW = NC * NS
B = 16384
D = 64
DP = 2 * D
BPW = B // NW
CHUNK = 128
NCHUNK = BPW // CHUNK
NBLK = BPW // L


def _body(cidx, aidx, tidx, sent, cat, auth, title, wb, out,
          cidx_v, aidx_v, tidx_v, pc_v, pa_v, pt_v, sent_v, wb_v,
          bufc, bufa, buft, acc_v, sem_s, sem_g):
  wid = lax.axis_index("s") * NC + lax.axis_index("c")
  base = wid * BPW

  stage = []
  for j in range(NCHUNK):
    off = base + CHUNK * j
    stage.append(pltpu.async_copy(cidx.at[pl.ds(off, CHUNK)], cidx_v.at[j], sem_s))
    stage.append(pltpu.async_copy(aidx.at[pl.ds(off, CHUNK)], aidx_v.at[j], sem_s))
    stage.append(pltpu.async_copy(tidx.at[pl.ds(off, CHUNK)], tidx_v.at[j], sem_s))
  stage.append(pltpu.async_copy(sent.at[pl.ds(base, BPW)], sent_v, sem_s))
  stage.append(pltpu.async_copy(wb, wb_v, sem_s))
  for h in stage:
    h.wait()

  for idx_v, par_v in ((cidx_v, pc_v), (aidx_v, pa_v), (tidx_v, pt_v)):
    for j in range(NCHUNK):
      for s in range(CHUNK // L):
        v = idx_v[j, pl.ds(L * s, L)]
        par_v[pl.ds(CHUNK * j + L * s, L)] = lax.bitwise_and(v, 1) * D
        idx_v[j, pl.ds(L * s, L)] = lax.shift_right_logical(v, 1)

  gath = []
  for j in range(NCHUNK):
    sl = pl.ds(CHUNK * j, CHUNK)
    gath.append(pltpu.async_copy(cat.at[cidx_v.at[j]], bufc.at[sl], sem_g))
    gath.append(pltpu.async_copy(auth.at[aidx_v.at[j]], bufa.at[sl], sem_g))
    gath.append(pltpu.async_copy(title.at[tidx_v.at[j]], buft.at[sl], sem_g))

  wlo = [wb_v[pl.ds(32 * g, L)] for g in range(6)]
  whi = [wb_v[pl.ds(32 * g + L, L)] for g in range(6)]
  wsv = (wb_v[pl.ds(192, L)] + wb_v[pl.ds(208, L)]) + (
      wb_v[pl.ds(224, L)] + wb_v[pl.ds(240, L)])
  ws_sum = jnp.sum(wsv)
  b_s = jnp.sum(wb_v[pl.ds(256, L)])

  for h in gath:
    h.wait()

  lane = lax.iota(jnp.int32, L)
  bufs = (bufc, bufa, buft)
  pars = (pc_v, pa_v, pt_v)
  himask = jnp.full((L,), -65536, jnp.int32)

  def blk(i, carry):
    r0 = i * L
    offv = [p[pl.ds(r0, L)] for p in pars]
    dots = jnp.zeros((L,), jnp.float32)
    for j in range(L):
      row = r0 + j
      pa = jnp.zeros((L,), jnp.float32)
      pb = jnp.zeros((L,), jnp.float32)
      for t in range(3):
        o = offv[t][j]
        for h in range(2):
          g = 2 * t + h
          v32 = bufs[t][row, pl.ds(o + 32 * h, 32)]
          pairs = plsc.bitcast(v32, jnp.int32)
          f_lo = plsc.bitcast(lax.shift_left(pairs, 16), jnp.float32)
          f_hi = plsc.bitcast(lax.bitwise_and(pairs, himask), jnp.float32)
          pa = pa + f_lo * wlo[g]
          pb = pb + f_hi * whi[g]
      sj = jnp.sum(pa + pb)
      dots = jnp.where(lane == j, sj, dots)
    acc_v[pl.ds(r0, L)] = dots + (sent_v[pl.ds(r0, L)] * ws_sum + b_s)
    return carry

  lax.fori_loop(0, NBLK, blk, 0)
  pltpu.sync_copy(acc_v, out.at[pl.ds(base, BPW)])


@functools.cache
def _build():
  mesh = plsc.VectorSubcoreMesh(
      core_axis_name="c", subcore_axis_name="s", num_cores=NC, num_subcores=NS)
  return pl.kernel(
      _body,
      out_type=jax.ShapeDtypeStruct((B,), jnp.float32),
      mesh=mesh,
      compiler_params=pltpu.CompilerParams(
          needs_layout_passes=False, use_tc_tiling_on_sc=False),
      scratch_types=[
          pltpu.VMEM((NCHUNK, CHUNK), jnp.int32),
          pltpu.VMEM((NCHUNK, CHUNK), jnp.int32),
          pltpu.VMEM((NCHUNK, CHUNK), jnp.int32),
          pltpu.VMEM((BPW,), jnp.int32),
          pltpu.VMEM((BPW,), jnp.int32),
          pltpu.VMEM((BPW,), jnp.int32),
          pltpu.VMEM((BPW,), jnp.float32),
          pltpu.VMEM((272,), jnp.float32),
          pltpu.VMEM((BPW, DP), jnp.bfloat16),
          pltpu.VMEM((BPW, DP), jnp.bfloat16),
          pltpu.VMEM((BPW, DP), jnp.bfloat16),
          pltpu.VMEM((BPW,), jnp.float32),
          pltpu.SemaphoreType.DMA,
          pltpu.SemaphoreType.DMA,
      ],
  )


def _paired_bf16(x):
  return x.astype(jnp.bfloat16).reshape(x.shape[0] // 2, 2 * x.shape[1])


def kernel(category_indices, author_indices, title_indices, sentiment_scores,
           category_table, author_table, title_table, linear_w, linear_b):
  w = linear_w.reshape(-1)
  wtab = w[:192].reshape(6, 16, 2).transpose(0, 2, 1).reshape(192)
  wb = jnp.concatenate([
      wtab, w[192:], linear_b.reshape(-1),
      jnp.zeros((15,), jnp.float32)])
  out = _build()(category_indices, author_indices, title_indices,
                 sentiment_scores, _paired_bf16(category_table),
                 _paired_bf16(author_table), _paired_bf16(title_table), wb)
  return out.reshape(B, 1)


# speedup vs baseline: 1.2854x; 1.2854x over previous
import functools

import jax
import jax.numpy as jnp
from jax import lax
from jax.experimental import pallas as pl
from jax.experimental.pallas import tpu as pltpu
from jax.experimental.pallas import tpu_sc as plsc

NC = 2
NS = 16
L = 16
NW = NC * NS
B = 16384
D = 64
DP = 2 * D
BPW = B // NW
CHUNK = 128
NCHUNK = BPW // CHUNK
NBLK = BPW // L


def _body(cidx, aidx, tidx, sent, cat, auth, title, wb, out,
          cidx_v, aidx_v, tidx_v, pc_v, pa_v, pt_v, sent_v, wb_v,
          rows, acc_v, sem_s, sem_g):
  wid = lax.axis_index("s") * NC + lax.axis_index("c")
  base = wid * BPW

  stage = []
  for j in range(NCHUNK):
    off = base + CHUNK * j
    stage.append(pltpu.async_copy(cidx.at[pl.ds(off, CHUNK)], cidx_v.at[j], sem_s))
    stage.append(pltpu.async_copy(aidx.at[pl.ds(off, CHUNK)], aidx_v.at[j], sem_s))
    stage.append(pltpu.async_copy(tidx.at[pl.ds(off, CHUNK)], tidx_v.at[j], sem_s))
  stage.append(pltpu.async_copy(sent.at[pl.ds(base, BPW)], sent_v, sem_s))
  stage.append(pltpu.async_copy(wb, wb_v, sem_s))
  for h in stage:
    h.wait()

  for idx_v, par_v in ((cidx_v, pc_v), (aidx_v, pa_v), (tidx_v, pt_v)):
    for j in range(NCHUNK):
      for s in range(CHUNK // L):
        v = idx_v[j, pl.ds(L * s, L)]
        par_v[pl.ds(CHUNK * j + L * s, L)] = lax.bitwise_and(v, 1) * D
        idx_v[j, pl.ds(L * s, L)] = lax.shift_right_logical(v, 1)

  wks = [wb_v[pl.ds(L * g, L)] for g in range(12)]
  wsv = (wb_v[pl.ds(192, L)] + wb_v[pl.ds(208, L)]) + (
      wb_v[pl.ds(224, L)] + wb_v[pl.ds(240, L)])
  ws_sum = jnp.sum(wsv)
  b_s = jnp.sum(wb_v[pl.ds(256, L)])

  def init_blk(i, carry):
    r0 = i * L
    acc_v[pl.ds(r0, L)] = sent_v[pl.ds(r0, L)] * ws_sum + b_s
    return carry
  lax.fori_loop(0, NBLK, init_blk, 0)

  lane = lax.iota(jnp.int32, L)

  for t, (table, idx_v, par_v) in enumerate((
      (cat, cidx_v, pc_v), (auth, aidx_v, pa_v), (title, tidx_v, pt_v))):
    gath = [
        pltpu.async_copy(table.at[idx_v.at[j]],
                         rows.at[pl.ds(CHUNK * j, CHUNK)], sem_g)
        for j in range(NCHUNK)
    ]
    for h in gath:
      h.wait()

    def blk(i, carry):
      r0 = i * L
      offv = par_v[pl.ds(r0, L)]
      dots = jnp.zeros((L,), jnp.float32)
      for j in range(L):
        row = r0 + j
        o = offv[j]
        pa = rows[row, pl.ds(o, L)] * wks[4 * t]
        pb = rows[row, pl.ds(o + L, L)] * wks[4 * t + 1]
        pa = pa + rows[row, pl.ds(o + 2 * L, L)] * wks[4 * t + 2]
        pb = pb + rows[row, pl.ds(o + 3 * L, L)] * wks[4 * t + 3]
        sj = jnp.sum(pa + pb)
        dots = jnp.where(lane == j, sj, dots)
      acc_v[pl.ds(r0, L)] = acc_v[pl.ds(r0, L)] + dots
      return carry

    lax.fori_loop(0, NBLK, blk, 0)

  pltpu.sync_copy(acc_v, out.at[pl.ds(base, BPW)])


@functools.cache
def _build():
  mesh = plsc.VectorSubcoreMesh(
      core_axis_name="c", subcore_axis_name="s", num_cores=NC, num_subcores=NS)
  return pl.kernel(
      _body,
      out_type=jax.ShapeDtypeStruct((B,), jnp.float32),
      mesh=mesh,
      compiler_params=pltpu.CompilerParams(
          needs_layout_passes=False, use_tc_tiling_on_sc=True),
      scratch_types=[
          pltpu.VMEM((NCHUNK, CHUNK), jnp.int32),
          pltpu.VMEM((NCHUNK, CHUNK), jnp.int32),
          pltpu.VMEM((NCHUNK, CHUNK), jnp.int32),
          pltpu.VMEM((BPW,), jnp.int32),
          pltpu.VMEM((BPW,), jnp.int32),
          pltpu.VMEM((BPW,), jnp.int32),
          pltpu.VMEM((BPW,), jnp.float32),
          pltpu.VMEM((272,), jnp.float32),
          pltpu.VMEM((BPW, DP), jnp.float32),
          pltpu.VMEM((BPW,), jnp.float32),
          pltpu.SemaphoreType.DMA,
          pltpu.SemaphoreType.DMA,
      ],
  )


def _paired(x):
  return x.reshape(x.shape[0] // 2, 2 * x.shape[1])


def kernel(category_indices, author_indices, title_indices, sentiment_scores,
           category_table, author_table, title_table, linear_w, linear_b):
  wb = jnp.concatenate([
      linear_w.reshape(-1), linear_b.reshape(-1),
      jnp.zeros((15,), jnp.float32)])
  out = _build()(category_indices, author_indices, title_indices,
                 sentiment_scores, _paired(category_table),
                 _paired(author_table), _paired(title_table), wb)
  return out.reshape(B, 1)
